# baseline (speedup 1.0000x reference)
import functools

import jax
import jax.numpy as jnp
from jax import lax
from jax.experimental import pallas as pl
from jax.experimental.pallas import tpu as pltpu
from jax.experimental.pallas import tpu_sc as plsc

BS = 16384
V = 100000
D = 64
IN = 4
H = 32
DP = 2 * D

_INFO = plsc.get_sparse_core_info()
_NC, _NS, _L = _INFO.num_cores, _INFO.num_subcores, _INFO.num_lanes
_NW = _NC * _NS
_BPW = BS // _NW
_IDX_MINOR = 128
_NG = _BPW // _IDX_MINOR


def _sc_gather(joint_token, emb2):
    @functools.partial(
        pl.kernel,
        out_type=jax.ShapeDtypeStruct((BS, DP), jnp.float32),
        mesh=plsc.VectorSubcoreMesh(core_axis_name="c", subcore_axis_name="s"),
        scratch_types=[
            pltpu.VMEM((_NG, _IDX_MINOR), jnp.int32),
            pltpu.VMEM((_BPW, DP), jnp.float32),
            pltpu.SemaphoreType.DMA,
        ],
    )
    def body(tok_hbm, emb_hbm, out_hbm, idx_v, rows_v, sem):
        wid = lax.axis_index("s") * _NC + lax.axis_index("c")
        base = wid * _BPW
        for j in range(_NG):
            pltpu.sync_copy(
                tok_hbm.at[pl.ds(base + j * _IDX_MINOR, _IDX_MINOR)],
                idx_v.at[j],
            )
        for g in range(_BPW // _L):
            j, off = divmod(g * _L, _IDX_MINOR)
            idx_v[j, pl.ds(off, _L)] = idx_v[j, pl.ds(off, _L)] >> 1
        copies = [
            pltpu.async_copy(
                emb_hbm.at[idx_v.at[j]],
                rows_v.at[pl.ds(j * _IDX_MINOR, _IDX_MINOR)],
                sem,
            )
            for j in range(_NG)
        ]
        for c in copies:
            c.wait()
        pltpu.sync_copy(rows_v, out_hbm.at[pl.ds(base, _BPW)])

    return body(joint_token, emb2)


_TC_ROWS = 2048


def _tc_body(info_ref, tok_ref, g_ref, pad_ref, w1t_ref, b1_ref, w2t_ref, o_ref):
    info = info_ref[...]
    h = jnp.dot(info, w1t_ref[...], preferred_element_type=jnp.float32)
    h = h + b1_ref[...]
    h = 0.5 * h * (1.0 + lax.erf(h * 0.7071067811865476))
    out = jnp.dot(h, w2t_ref[...], preferred_element_type=jnp.float32)
    gpair = g_ref[...]
    parity = (tok_ref[...] & 1) == 1
    sel = jnp.where(parity, gpair[:, D:], gpair[:, :D])
    mask = jnp.max(jnp.abs(info), axis=1, keepdims=True) == 0.0
    pad_emb = pad_ref[...][0:1, :D]
    o_ref[...] = jnp.where(mask, pad_emb, out + sel)


def _tc_mlp_add(joint_info, joint_token, gathered, emb2, W1, b1, W2):
    grid = BS // _TC_ROWS
    return pl.pallas_call(
        _tc_body,
        grid=(grid,),
        in_specs=[
            pl.BlockSpec((_TC_ROWS, IN), lambda i: (i, 0)),
            pl.BlockSpec((_TC_ROWS, 1), lambda i: (i, 0)),
            pl.BlockSpec((_TC_ROWS, DP), lambda i: (i, 0)),
            pl.BlockSpec((8, DP), lambda i: (0, 0)),
            pl.BlockSpec((IN, H), lambda i: (0, 0)),
            pl.BlockSpec((1, H), lambda i: (0, 0)),
            pl.BlockSpec((H, D), lambda i: (0, 0)),
        ],
        out_specs=pl.BlockSpec((_TC_ROWS, D), lambda i: (i, 0)),
        out_shape=jax.ShapeDtypeStruct((BS, D), jnp.float32),
    )(
        joint_info,
        joint_token.reshape(BS, 1),
        gathered,
        emb2,
        W1.T,
        b1.reshape(1, H),
        W2.T,
    )


def kernel(joint_info, joint_token, emb, W1, b1, W2):
    tok = joint_token.astype(jnp.int32)
    emb2 = emb.reshape(V // 2, DP)
    gathered = _sc_gather(tok, emb2)
    return _tc_mlp_add(joint_info, tok, gathered, emb2, W1, b1, W2)


# speedup vs baseline: 1.1863x; 1.1863x over previous
import functools

import jax
import jax.numpy as jnp
from jax import lax
from jax.experimental import pallas as pl
from jax.experimental.pallas import tpu as pltpu
from jax.experimental.pallas import tpu_sc as plsc

BS = 16384
V = 100000
D = 64
IN = 4
H = 32
DP = 2 * D

_INFO = plsc.get_sparse_core_info()
_NC, _NS, _L = _INFO.num_cores, _INFO.num_subcores, _INFO.num_lanes
_NW = _NC * _NS
_BPW = BS // _NW
_IDX_MINOR = 128
_NG = _BPW // _IDX_MINOR

_RP_COLS = 1024
_NBLK = 49
_SPLIT = _NBLK * _RP_COLS


def _repack_body(a_ref, b_ref, o_ref):
    a = a_ref[...]
    b = b_ref[...]
    o_ref[...] = jnp.concatenate([a.T, b.T], axis=1)


def _tc_repack(embt):
    return pl.pallas_call(
        _repack_body,
        grid=(_NBLK,),
        in_specs=[
            pl.BlockSpec((D, _RP_COLS), lambda i: (0, i)),
            pl.BlockSpec((D, _RP_COLS), lambda i: (0, i + _NBLK)),
        ],
        out_specs=pl.BlockSpec((_RP_COLS, DP), lambda i: (i, 0)),
        out_shape=jax.ShapeDtypeStruct((_SPLIT, DP), jnp.float32),
    )(embt, embt)


def _sc_gather(joint_token, emb128):
    @functools.partial(
        pl.kernel,
        out_type=jax.ShapeDtypeStruct((BS, DP), jnp.float32),
        mesh=plsc.VectorSubcoreMesh(core_axis_name="c", subcore_axis_name="s"),
        scratch_types=[
            pltpu.VMEM((_NG, _IDX_MINOR), jnp.int32),
            pltpu.VMEM((_BPW, DP), jnp.float32),
            pltpu.SemaphoreType.DMA,
        ],
    )
    def body(tok_hbm, emb_hbm, out_hbm, idx_v, rows_v, sem):
        wid = lax.axis_index("s") * _NC + lax.axis_index("c")
        base = wid * _BPW
        for j in range(_NG):
            pltpu.sync_copy(
                tok_hbm.at[pl.ds(base + j * _IDX_MINOR, _IDX_MINOR)],
                idx_v.at[j],
            )
        for g in range(_BPW // _L):
            j, off = divmod(g * _L, _IDX_MINOR)
            t = idx_v[j, pl.ds(off, _L)]
            idx_v[j, pl.ds(off, _L)] = jnp.where(t >= _SPLIT, t - _SPLIT, t)
        copies = [
            pltpu.async_copy(
                emb_hbm.at[idx_v.at[j]],
                rows_v.at[pl.ds(j * _IDX_MINOR, _IDX_MINOR)],
                sem,
            )
            for j in range(_NG)
        ]
        for c in copies:
            c.wait()
        pltpu.sync_copy(rows_v, out_hbm.at[pl.ds(base, _BPW)])

    return body(joint_token, emb128)


_TC_ROWS = 2048


def _tc_body(info_ref, tok_ref, g_ref, pad_ref, w1t_ref, b1_ref, w2t_ref, o_ref):
    info = info_ref[...]
    h = jnp.dot(info, w1t_ref[...], preferred_element_type=jnp.float32)
    h = h + b1_ref[...]
    h = 0.5 * h * (1.0 + lax.erf(h * 0.7071067811865476))
    out = jnp.dot(h, w2t_ref[...], preferred_element_type=jnp.float32)
    gpair = g_ref[...]
    hi = tok_ref[...] >= _SPLIT
    sel = jnp.where(hi, gpair[:, D:], gpair[:, :D])
    mask = jnp.max(jnp.abs(info), axis=1, keepdims=True) == 0.0
    pad_emb = pad_ref[...][0:1, :D]
    o_ref[...] = jnp.where(mask, pad_emb, out + sel)


def _tc_mlp_add(joint_info, joint_token, gathered, emb2, W1, b1, W2):
    grid = BS // _TC_ROWS
    return pl.pallas_call(
        _tc_body,
        grid=(grid,),
        in_specs=[
            pl.BlockSpec((_TC_ROWS, IN), lambda i: (i, 0)),
            pl.BlockSpec((_TC_ROWS, 1), lambda i: (i, 0)),
            pl.BlockSpec((_TC_ROWS, DP), lambda i: (i, 0)),
            pl.BlockSpec((8, DP), lambda i: (0, 0)),
            pl.BlockSpec((IN, H), lambda i: (0, 0)),
            pl.BlockSpec((1, H), lambda i: (0, 0)),
            pl.BlockSpec((H, D), lambda i: (0, 0)),
        ],
        out_specs=pl.BlockSpec((_TC_ROWS, D), lambda i: (i, 0)),
        out_shape=jax.ShapeDtypeStruct((BS, D), jnp.float32),
    )(
        joint_info,
        joint_token.reshape(BS, 1),
        gathered,
        emb2,
        W1.T,
        b1.reshape(1, H),
        W2.T,
    )


def kernel(joint_info, joint_token, emb, W1, b1, W2):
    tok = joint_token.astype(jnp.int32)
    emb2 = _tc_repack(emb.T)
    gathered = _sc_gather(tok, emb2)
    return _tc_mlp_add(joint_info, tok, gathered, emb2, W1, b1, W2)


# speedup vs baseline: 1.3947x; 1.1756x over previous
import functools

import jax
import jax.numpy as jnp
from jax import lax
from jax.experimental import pallas as pl
from jax.experimental.pallas import tpu as pltpu
from jax.experimental.pallas import tpu_sc as plsc

BS = 16384
V = 100000
D = 64
IN = 4
H = 32
DP = 2 * D

_INFO = plsc.get_sparse_core_info()
_NC, _NS, _L = _INFO.num_cores, _INFO.num_subcores, _INFO.num_lanes
_NW = _NC * _NS
_BPW = BS // _NW
_IDX_MINOR = 128
_NG = _BPW // _IDX_MINOR

_RP_COLS = 1024
_NBLK = 49
_SPLIT = _NBLK * _RP_COLS


def _t(x, ident):
    return lax.dot_general(
        x, ident, (((0,), (0,)), ((), ())),
        preferred_element_type=jnp.float32,
    )


def _repack_body(a_ref, b_ref, i64_ref, o_ref):
    ident = i64_ref[...]
    a_t = _t(a_ref[...], ident)
    b_t = _t(b_ref[...], ident)
    o_ref[...] = jnp.concatenate([a_t, b_t], axis=1)


def _tc_repack(embt, i64):
    return pl.pallas_call(
        _repack_body,
        grid=(_NBLK,),
        in_specs=[
            pl.BlockSpec((D, _RP_COLS), lambda i: (0, i)),
            pl.BlockSpec((D, _RP_COLS), lambda i: (0, i + _NBLK)),
            pl.BlockSpec((D, D), lambda i: (0, 0)),
        ],
        out_specs=pl.BlockSpec((_RP_COLS, DP), lambda i: (i, 0)),
        out_shape=jax.ShapeDtypeStruct((_SPLIT, DP), jnp.float32),
    )(embt, embt, i64)


def _sc_gather(joint_token, emb2):
    @functools.partial(
        pl.kernel,
        out_type=jax.ShapeDtypeStruct((BS, DP), jnp.float32),
        mesh=plsc.VectorSubcoreMesh(core_axis_name="c", subcore_axis_name="s"),
        scratch_types=[
            pltpu.VMEM((_NG, _IDX_MINOR), jnp.int32),
            pltpu.VMEM((_BPW, DP), jnp.float32),
            pltpu.SemaphoreType.DMA,
        ],
    )
    def body(tok_hbm, emb_hbm, out_hbm, idx_v, rows_v, sem):
        wid = lax.axis_index("s") * _NC + lax.axis_index("c")
        base = wid * _BPW
        for j in range(_NG):
            pltpu.sync_copy(
                tok_hbm.at[pl.ds(base + j * _IDX_MINOR, _IDX_MINOR)],
                idx_v.at[j],
            )
        for g in range(_BPW // _L):
            j, off = divmod(g * _L, _IDX_MINOR)
            t = idx_v[j, pl.ds(off, _L)]
            idx_v[j, pl.ds(off, _L)] = jnp.where(t >= _SPLIT, t - _SPLIT, t)
        copies = [
            pltpu.async_copy(
                emb_hbm.at[idx_v.at[j]],
                rows_v.at[pl.ds(j * _IDX_MINOR, _IDX_MINOR)],
                sem,
            )
            for j in range(_NG)
        ]
        for c in copies:
            c.wait()
        pltpu.sync_copy(rows_v, out_hbm.at[pl.ds(base, _BPW)])

    return body(joint_token, emb2)


_TC_ROWS = 2048
_TOK_ROWS = BS // _TC_ROWS


def _tc_body(infot_ref, tok_ref, g_ref, pad_ref, w1t_ref, b1_ref, w2t_ref,
             i64_ref, o_ref):
    i = pl.program_id(0)
    info_t = infot_ref[...]
    h = lax.dot_general(
        w1t_ref[...], info_t, (((0,), (0,)), ((), ())),
        preferred_element_type=jnp.float32,
    )
    h = h + b1_ref[...].T
    h = 0.5 * h * (1.0 + lax.erf(h * 0.7071067811865476))
    mlp = lax.dot_general(
        w2t_ref[...], h, (((0,), (0,)), ((), ())),
        preferred_element_type=jnp.float32,
    )
    ident = i64_ref[...]
    gpair = g_ref[...]
    lo_t = lax.dot_general(
        ident, gpair[:, :D], (((0,), (1,)), ((), ())),
        preferred_element_type=jnp.float32,
    )
    hi_t = lax.dot_general(
        ident, gpair[:, D:], (((0,), (1,)), ((), ())),
        preferred_element_type=jnp.float32,
    )
    tok_row = tok_ref[pl.ds(i, 1), :]
    sel = jnp.where(tok_row >= _SPLIT, hi_t, lo_t)
    mask = jnp.max(jnp.abs(info_t), axis=0, keepdims=True) == 0.0
    pad_col = pad_ref[...][0:1, :D].T
    o_ref[...] = jnp.where(mask, pad_col, mlp + sel)


def _tc_mlp_add(infot, tok2d, gathered, emb2, W1, b1, W2, i64):
    grid = BS // _TC_ROWS
    return pl.pallas_call(
        _tc_body,
        grid=(grid,),
        in_specs=[
            pl.BlockSpec((IN, _TC_ROWS), lambda i: (0, i)),
            pl.BlockSpec((_TOK_ROWS, _TC_ROWS), lambda i: (0, 0)),
            pl.BlockSpec((_TC_ROWS, DP), lambda i: (i, 0)),
            pl.BlockSpec((8, DP), lambda i: (0, 0)),
            pl.BlockSpec((IN, H), lambda i: (0, 0)),
            pl.BlockSpec((1, H), lambda i: (0, 0)),
            pl.BlockSpec((H, D), lambda i: (0, 0)),
            pl.BlockSpec((D, D), lambda i: (0, 0)),
        ],
        out_specs=pl.BlockSpec((D, _TC_ROWS), lambda i: (0, i)),
        out_shape=jax.ShapeDtypeStruct((D, BS), jnp.float32),
    )(
        infot,
        tok2d,
        gathered,
        emb2,
        W1.T,
        b1.reshape(1, H),
        W2.T,
        i64,
    )


def kernel(joint_info, joint_token, emb, W1, b1, W2):
    tok = joint_token.astype(jnp.int32)
    i64 = jnp.eye(D, dtype=jnp.float32)
    emb2 = _tc_repack(emb.T, i64)
    gathered = _sc_gather(tok, emb2)
    out_t = _tc_mlp_add(
        joint_info.T,
        tok.reshape(_TOK_ROWS, _TC_ROWS),
        gathered,
        emb2,
        W1,
        b1,
        W2,
        i64,
    )
    return out_t.T


# speedup vs baseline: 1.6503x; 1.1833x over previous
import functools

import jax
import jax.numpy as jnp
from jax import lax
from jax.experimental import pallas as pl
from jax.experimental.pallas import tpu as pltpu
from jax.experimental.pallas import tpu_sc as plsc

BS = 16384
V = 100000
D = 64
IN = 4
H = 32
DP = 2 * D

_INFO = plsc.get_sparse_core_info()
_NC, _NS, _L = _INFO.num_cores, _INFO.num_subcores, _INFO.num_lanes
_NW = _NC * _NS
_BPW = BS // _NW
_IDX_MINOR = 128
_NG = _BPW // _IDX_MINOR

_RP_COLS = 2048
_NBLK = 25
_SPLIT = _NBLK * _RP_COLS
_BMAX = V // _RP_COLS


def _t(x, ident):
    return lax.dot_general(
        x, ident, (((0,), (0,)), ((), ())),
        preferred_element_type=jnp.float32,
    )


def _repack_body(a_ref, b_ref, i64_ref, o_ref):
    ident = i64_ref[...]
    a_t = _t(a_ref[...], ident)
    b_t = _t(b_ref[...], ident)
    o_ref[...] = jnp.concatenate([a_t, b_t], axis=1)


def _tc_repack(embt, i64):
    return pl.pallas_call(
        _repack_body,
        grid=(_NBLK,),
        in_specs=[
            pl.BlockSpec((D, _RP_COLS), lambda i: (0, i)),
            pl.BlockSpec((D, _RP_COLS), lambda i: (0, jnp.minimum(i + _NBLK, _BMAX))),
            pl.BlockSpec((D, D), lambda i: (0, 0)),
        ],
        out_specs=pl.BlockSpec((_RP_COLS, DP), lambda i: (i, 0)),
        out_shape=jax.ShapeDtypeStruct((_SPLIT, DP), jnp.float32),
    )(embt, embt, i64)


def _sc_gather(joint_token, emb2):
    @functools.partial(
        pl.kernel,
        out_type=jax.ShapeDtypeStruct((BS, DP), jnp.float32),
        mesh=plsc.VectorSubcoreMesh(core_axis_name="c", subcore_axis_name="s"),
        scratch_types=[
            pltpu.VMEM((_NG, _IDX_MINOR), jnp.int32),
            pltpu.VMEM((_BPW, DP), jnp.float32),
            pltpu.SemaphoreType.DMA,
        ],
    )
    def body(tok_hbm, emb_hbm, out_hbm, idx_v, rows_v, sem):
        wid = lax.axis_index("s") * _NC + lax.axis_index("c")
        base = wid * _BPW
        for j in range(_NG):
            pltpu.sync_copy(
                tok_hbm.at[pl.ds(base + j * _IDX_MINOR, _IDX_MINOR)],
                idx_v.at[j],
            )
        for g in range(_BPW // _L):
            j, off = divmod(g * _L, _IDX_MINOR)
            t = idx_v[j, pl.ds(off, _L)]
            idx_v[j, pl.ds(off, _L)] = jnp.where(t >= _SPLIT, t - _SPLIT, t)
        copies = [
            pltpu.async_copy(
                emb_hbm.at[idx_v.at[j]],
                rows_v.at[pl.ds(j * _IDX_MINOR, _IDX_MINOR)],
                sem,
            )
            for j in range(_NG)
        ]
        for c in copies:
            c.wait()
        pltpu.sync_copy(rows_v, out_hbm.at[pl.ds(base, _BPW)])

    return body(joint_token, emb2)


_TC_ROWS = 2048
_TOK_ROWS = BS // _TC_ROWS


def _tc_body(infot_ref, tok_ref, g_ref, pad_ref, w1t_ref, b1_ref, w2t_ref,
             i64_ref, o_ref):
    i = pl.program_id(0)
    info_t = infot_ref[...]
    h = lax.dot_general(
        w1t_ref[...], info_t, (((0,), (0,)), ((), ())),
        preferred_element_type=jnp.float32,
    )
    h = h + b1_ref[...].T
    h = 0.5 * h * (1.0 + lax.erf(h * 0.7071067811865476))
    mlp = lax.dot_general(
        w2t_ref[...], h, (((0,), (0,)), ((), ())),
        preferred_element_type=jnp.float32,
    )
    ident = i64_ref[...]
    gpair = g_ref[...]
    lo_t = lax.dot_general(
        ident, gpair[:, :D], (((0,), (1,)), ((), ())),
        preferred_element_type=jnp.float32,
    )
    hi_t = lax.dot_general(
        ident, gpair[:, D:], (((0,), (1,)), ((), ())),
        preferred_element_type=jnp.float32,
    )
    tok_row = tok_ref[pl.ds(i, 1), :]
    sel = jnp.where(tok_row >= _SPLIT, hi_t, lo_t)
    mask = jnp.max(jnp.abs(info_t), axis=0, keepdims=True) == 0.0
    pad_col = pad_ref[...][0:1, :D].T
    o_ref[...] = jnp.where(mask, pad_col, mlp + sel)


def _tc_mlp_add(infot, tok2d, gathered, emb2, W1, b1, W2, i64):
    grid = BS // _TC_ROWS
    return pl.pallas_call(
        _tc_body,
        grid=(grid,),
        in_specs=[
            pl.BlockSpec((IN, _TC_ROWS), lambda i: (0, i)),
            pl.BlockSpec((_TOK_ROWS, _TC_ROWS), lambda i: (0, 0)),
            pl.BlockSpec((_TC_ROWS, DP), lambda i: (i, 0)),
            pl.BlockSpec((8, DP), lambda i: (0, 0)),
            pl.BlockSpec((IN, H), lambda i: (0, 0)),
            pl.BlockSpec((1, H), lambda i: (0, 0)),
            pl.BlockSpec((H, D), lambda i: (0, 0)),
            pl.BlockSpec((D, D), lambda i: (0, 0)),
        ],
        out_specs=pl.BlockSpec((D, _TC_ROWS), lambda i: (0, i)),
        out_shape=jax.ShapeDtypeStruct((D, BS), jnp.float32),
    )(
        infot,
        tok2d,
        gathered,
        emb2,
        W1.T,
        b1.reshape(1, H),
        W2.T,
        i64,
    )


def kernel(joint_info, joint_token, emb, W1, b1, W2):
    tok = joint_token.astype(jnp.int32)
    i64 = jnp.eye(D, dtype=jnp.float32)
    emb2 = _tc_repack(emb.T, i64)
    gathered = _sc_gather(tok, emb2)
    out_t = _tc_mlp_add(
        joint_info.T,
        tok.reshape(_TOK_ROWS, _TC_ROWS),
        gathered,
        emb2,
        W1,
        b1,
        W2,
        i64,
    )
    return out_t.T
